# Initial kernel scaffold; baseline (speedup 1.0000x reference)
import jax
import jax.numpy as jnp
from jax.experimental import pallas as pl


def kernel(page_hash, offset, cache_line, addr_alignment, stride, reuse_dist, locality_cluster, entropy, address_flags, tbl_ph, tbl_off, tbl_cl, tbl_aa, tbl_str, tbl_rd, tbl_lc, tbl_ent, W_flags, b_flags, W_gate, b_gate, W_tr, b_tr, gamma, beta):
    raise NotImplementedError("write your pallas kernel here")


# speedup vs baseline: 6.0813x; 6.0813x over previous
import functools

import jax
import jax.numpy as jnp
from jax import lax
from jax.experimental import pallas as pl
from jax.experimental.pallas import tpu as pltpu
from jax.experimental.pallas import tpu_sc as plsc

_D = (8, 4, 4, 3, 6, 5, 6, 3)
_OFF = (0, 8, 12, 16, 19, 25, 30, 36)
_NFEAT = 39
_FLAG_ROW = 40
_TR = 48
_NC, _NS = 2, 16
_NW = _NC * _NS


@functools.lru_cache(maxsize=2)
def _sc_gather(n_tokens):
    nt = n_tokens // 128
    tiles_per_w = nt // _NW
    mesh = plsc.VectorSubcoreMesh(
        core_axis_name="c", subcore_axis_name="s",
        num_cores=_NC, num_subcores=_NS)

    def body(*refs):
        idx_refs = refs[0:8]
        flags_ref = refs[8]
        tbl_refs = refs[9:17]
        out_ref = refs[17]
        ibufs = refs[18:26]
        gbufs = refs[26:34]
        fbuf = refs[34]
        tile = refs[35]
        sem = refs[36]

        wid = lax.axis_index("s") * _NC + lax.axis_index("c")
        iota = lax.iota(jnp.int32, 16)
        zeros = jnp.zeros((16,), jnp.float32)

        def step(t, carry):
            gt = wid * tiles_per_w + t
            base = gt * 128
            for i in range(8):
                pltpu.sync_copy(idx_refs[i].at[pl.ds(base, 128)], ibufs[i])
            pltpu.sync_copy(flags_ref.at[pl.ds(base * 5, 640)], fbuf)
            copies = [
                pltpu.async_copy(tbl_refs[i].at[ibufs[i]], gbufs[i], sem)
                for i in range(8)
            ]
            for c in copies:
                c.wait()
            for lg in range(8):
                rows = iota + lg * 16
                for i in range(8):
                    for k in range(_D[i]):
                        col = jnp.full((16,), k, jnp.int32)
                        v = plsc.load_gather(gbufs[i], [rows, col])
                        tile[pl.ds((_OFF[i] + k) * 128 + lg * 16, 16)] = v
                fbase = iota * 5 + lg * 80
                for k in range(5):
                    v = plsc.load_gather(fbuf, [fbase + k])
                    tile[pl.ds((_FLAG_ROW + k) * 128 + lg * 16, 16)] = v
                for r in (39, 45, 46, 47):
                    tile[pl.ds(r * 128 + lg * 16, 16)] = zeros
            pltpu.sync_copy(tile, out_ref.at[pl.ds(gt * (_TR * 128), _TR * 128)])
            return carry

        lax.fori_loop(0, tiles_per_w, step, 0)

    scratch = (
        [pltpu.VMEM((128,), jnp.int32) for _ in range(8)]
        + [pltpu.VMEM((128, 8), jnp.float32) for _ in _D]
        + [pltpu.VMEM((640,), jnp.float32),
           pltpu.VMEM((_TR * 128,), jnp.float32),
           pltpu.SemaphoreType.DMA]
    )
    return pl.kernel(
        body,
        out_type=jax.ShapeDtypeStruct((nt * _TR * 128,), jnp.float32),
        mesh=mesh,
        scratch_types=scratch,
        compiler_params=pltpu.CompilerParams(
            needs_layout_passes=False, use_tc_tiling_on_sc=False),
        name="sc_embed_gather",
    )


@functools.lru_cache(maxsize=2)
def _tc_dense(n_tokens):
    nt = n_tokens // 128
    grp = 8
    grid = nt // grp

    def body(ct_ref, wa_ref, v8_ref, wf8_ref, bf8_ref, bcat_ref,
             gam_ref, bet_ref, out_ref):
        wf = wf8_ref[...]
        scale = jnp.sum(jnp.abs(wf)) / 25.0 + 1e-8
        wq = jnp.clip(jnp.round(wf / scale), -1.0, 1.0) * scale
        v8 = v8_ref[...]
        dn = (((0,), (0,)), ((), ()))
        mq = lax.dot_general(wq, v8, dn, preferred_element_type=jnp.float32)
        bfv = lax.dot_general(bf8_ref[...], v8, (((1,), (0,)), ((), ())),
                              preferred_element_type=jnp.float32)
        wa = wa_ref[...]
        bias = bcat_ref[...] + bfv
        gam = gam_ref[...]
        bet = bet_ref[...]
        for g in range(grp):
            tile = ct_ref[g]
            big = lax.dot_general(tile, wa, dn,
                                  preferred_element_type=jnp.float32)
            fl8 = tile[_FLAG_ROW:_FLAG_ROW + 8, :]
            big = big + lax.dot_general(fl8, mq, dn,
                                        preferred_element_type=jnp.float32)
            big = big + bias
            gate = jax.nn.sigmoid(big[:, :64])
            z = gate * big[:, 64:]
            mu = jnp.mean(z, axis=-1, keepdims=True)
            d = z - mu
            var = jnp.mean(d * d, axis=-1, keepdims=True)
            o = d * lax.rsqrt(var + 1e-5) * gam + bet
            out_ref[pl.ds(g * 128, 128), :] = o

    return pl.pallas_call(
        body,
        grid=(grid,),
        in_specs=[
            pl.BlockSpec((grp, _TR, 128), lambda i: (i, 0, 0)),
            pl.BlockSpec((_TR, 128), lambda i: (0, 0)),
            pl.BlockSpec((8, 128), lambda i: (0, 0)),
            pl.BlockSpec((8, 8), lambda i: (0, 0)),
            pl.BlockSpec((1, 8), lambda i: (0, 0)),
            pl.BlockSpec((1, 128), lambda i: (0, 0)),
            pl.BlockSpec((1, 64), lambda i: (0, 0)),
            pl.BlockSpec((1, 64), lambda i: (0, 0)),
        ],
        out_specs=pl.BlockSpec((grp * 128, 64), lambda i: (i, 0)),
        out_shape=jax.ShapeDtypeStruct((n_tokens, 64), jnp.float32),
        compiler_params=pltpu.CompilerParams(
            dimension_semantics=("arbitrary",)),
        name="tc_fused_dense",
    )


def kernel(page_hash, offset, cache_line, addr_alignment, stride, reuse_dist,
           locality_cluster, entropy, address_flags, tbl_ph, tbl_off, tbl_cl,
           tbl_aa, tbl_str, tbl_rd, tbl_lc, tbl_ent, W_flags, b_flags,
           W_gate, b_gate, W_tr, b_tr, gamma, beta):
    bsz, seq = page_hash.shape
    n = bsz * seq
    idxs = [jnp.reshape(a, (n,)).astype(jnp.int32)
            for a in (page_hash, offset, cache_line, addr_alignment, stride,
                      reuse_dist, locality_cluster, entropy)]
    flags_flat = jnp.reshape(address_flags, (n * 5,))
    tbls = [tbl_ph, tbl_off, tbl_cl, tbl_aa, tbl_str, tbl_rd, tbl_lc, tbl_ent]
    tbls = [t if t.shape[1] == 8 else jnp.pad(t, ((0, 0), (0, 8 - t.shape[1])))
            for t in tbls]
    comb_flat = _sc_gather(n)(*idxs, flags_flat, *tbls)
    comb_t = comb_flat.reshape(n // 128, _TR, 128)

    wcat_t = jnp.concatenate([W_gate, W_tr], axis=0).T
    wa = jnp.zeros((_TR, 128), jnp.float32).at[:_NFEAT].set(wcat_t[:_NFEAT])
    v8 = jnp.zeros((8, 128), jnp.float32).at[:5].set(wcat_t[_NFEAT:44])
    wf8 = jnp.zeros((8, 8), jnp.float32).at[:5, :5].set(W_flags)
    bf8 = jnp.zeros((1, 8), jnp.float32).at[0, :5].set(b_flags)
    bcat = jnp.concatenate([b_gate, b_tr]).reshape(1, 128)
    out2 = _tc_dense(n)(comb_t, wa, v8, wf8, bf8, bcat,
                        gamma.reshape(1, 64), beta.reshape(1, 64))
    return out2.reshape(bsz, seq, 64)


# speedup vs baseline: 14.2300x; 2.3399x over previous
import functools

import jax
import jax.numpy as jnp
from jax import lax
from jax.experimental import pallas as pl
from jax.experimental.pallas import tpu as pltpu
from jax.experimental.pallas import tpu_sc as plsc

_D = (8, 4, 4, 3, 6, 5, 6, 3)
_OFF = (0, 8, 12, 16, 19, 25, 30, 36)
_NFEAT = 39
_FLAG_ROW = 40
_TR = 48
_NC, _NS = 2, 16
_NW = _NC * _NS


class _Slot:
    def __init__(self, ib, fb, gb, tile, sem_l, sem_g, sem_s):
        self.ib, self.fb, self.gb, self.tile = ib, fb, gb, tile
        self.sem_l, self.sem_g, self.sem_s = sem_l, sem_g, sem_s


@functools.lru_cache(maxsize=2)
def _sc_gather(n_tokens):
    nt = n_tokens // 128
    tiles_per_w = nt // _NW
    assert tiles_per_w % 2 == 0 and tiles_per_w >= 4
    mesh = plsc.VectorSubcoreMesh(
        core_axis_name="c", subcore_axis_name="s",
        num_cores=_NC, num_subcores=_NS)

    def body(idx_ref, flags_ref, t0, t1, t2, t3, t4, t5, t6, t7, out_ref,
             ib_a, fb_a, ga0, ga1, ga2, ga3, ga4, ga5, ga6, ga7, tile_a,
             ib_b, fb_b, gb0, gb1, gb2, gb3, gb4, gb5, gb6, gb7, tile_b,
             sem_la, sem_ga, sem_sa, sem_lb, sem_gb, sem_sb):
        tbl_refs = (t0, t1, t2, t3, t4, t5, t6, t7)
        sa = _Slot(ib_a, fb_a, (ga0, ga1, ga2, ga3, ga4, ga5, ga6, ga7),
                   tile_a, sem_la, sem_ga, sem_sa)
        sb = _Slot(ib_b, fb_b, (gb0, gb1, gb2, gb3, gb4, gb5, gb6, gb7),
                   tile_b, sem_lb, sem_gb, sem_sb)

        wid = lax.axis_index("s") * _NC + lax.axis_index("c")
        wbase = wid * tiles_per_w
        iota = lax.iota(jnp.int32, 16)
        zeros = jnp.zeros((16,), jnp.float32)
        cols = [jnp.full((16,), k, jnp.int32) for k in range(8)]

        def fire_load(s, gt):
            pltpu.async_copy(idx_ref.at[gt], s.ib, s.sem_l)
            pltpu.async_copy(flags_ref.at[pl.ds(gt * 640, 640)], s.fb, s.sem_l)

        def wait_load(s, gt):
            pltpu.make_async_copy(idx_ref.at[gt], s.ib, s.sem_l).wait()
            pltpu.make_async_copy(
                flags_ref.at[pl.ds(gt * 640, 640)], s.fb, s.sem_l).wait()

        def fire_gather(s):
            for i in range(8):
                pltpu.async_copy(tbl_refs[i].at[s.ib.at[i]], s.gb[i], s.sem_g)

        def wait_gather(s):
            for i in range(8):
                pltpu.make_async_copy(
                    tbl_refs[i].at[s.ib.at[i]], s.gb[i], s.sem_g).wait()

        def transpose_flags(s):
            for lg in range(8):
                fbase = iota * 5 + lg * 80
                for k in range(5):
                    v = plsc.load_gather(s.fb, [fbase + k])
                    s.tile[_FLAG_ROW + k, pl.ds(lg * 16, 16)] = v

        def transpose_emb(s):
            for lg in range(8):
                rows = iota + lg * 16
                for i in range(8):
                    for k in range(_D[i]):
                        v = plsc.load_gather(s.gb[i], [rows, cols[k]])
                        s.tile[_OFF[i] + k, pl.ds(lg * 16, 16)] = v

        def fire_store(s, gt):
            pltpu.async_copy(s.tile, out_ref.at[pl.ds(gt * _TR, _TR), :],
                             s.sem_s)

        def wait_store(s, gt_old):
            pltpu.make_async_copy(
                s.tile, out_ref.at[pl.ds(gt_old * _TR, _TR), :], s.sem_s
            ).wait()

        for s in (sa, sb):
            for r in (39, 45, 46, 47):
                for lg in range(8):
                    s.tile[r, pl.ds(lg * 16, 16)] = zeros

        fire_load(sa, wbase)
        wait_load(sa, wbase)
        fire_gather(sa)
        fire_load(sb, wbase + 1)

        def half(k, s, o, t_cur, t_nxt):
            wait_load(o, t_nxt)
            fire_gather(o)
            wait_gather(s)

            @pl.when(k > 0)
            def _():
                wait_store(s, t_cur - 2)

            transpose_flags(s)
            fire_load(s, t_nxt + 1)
            transpose_emb(s)
            fire_store(s, t_cur)

        def step(k, carry):
            t = wbase + 2 * k
            half(k, sa, sb, t, t + 1)
            half(k, sb, sa, t + 1, t + 2)
            return carry

        lax.fori_loop(0, tiles_per_w // 2 - 1, step, 0)

        t_last = wbase + tiles_per_w - 1
        wait_gather(sa)
        wait_store(sa, t_last - 3)
        transpose_flags(sa)
        transpose_emb(sa)
        fire_store(sa, t_last - 1)
        wait_load(sb, t_last)
        fire_gather(sb)
        wait_gather(sb)
        wait_store(sb, t_last - 2)
        transpose_flags(sb)
        transpose_emb(sb)
        fire_store(sb, t_last)
        wait_store(sa, t_last - 1)
        wait_store(sb, t_last)

    scratch = (
        [pltpu.VMEM((8, 128), jnp.int32), pltpu.VMEM((640,), jnp.float32)]
        + [pltpu.VMEM((128, 8), jnp.float32) for _ in range(8)]
        + [pltpu.VMEM((_TR, 128), jnp.float32)]
        + [pltpu.VMEM((8, 128), jnp.int32), pltpu.VMEM((640,), jnp.float32)]
        + [pltpu.VMEM((128, 8), jnp.float32) for _ in range(8)]
        + [pltpu.VMEM((_TR, 128), jnp.float32)]
        + [pltpu.SemaphoreType.DMA] * 6
    )
    return pl.kernel(
        body,
        out_type=jax.ShapeDtypeStruct((nt * _TR, 128), jnp.float32),
        mesh=mesh,
        scratch_types=scratch,
        compiler_params=pltpu.CompilerParams(
            needs_layout_passes=False, use_tc_tiling_on_sc=False),
        name="sc_embed_gather",
    )


@functools.lru_cache(maxsize=2)
def _tc_dense(bsz, seq):
    grp = 8
    bblk = bsz // (grp * 128)

    def body(ct_ref, wa_ref, v8_ref, wf8_ref, bf8_ref, bcat_ref,
             gam_ref, bet_ref, out_ref):
        wf = wf8_ref[...]
        scale = jnp.sum(jnp.abs(wf)) / 25.0 + 1e-8
        wq = jnp.clip(jnp.round(wf / scale), -1.0, 1.0) * scale
        v8 = v8_ref[...]
        dn = (((0,), (0,)), ((), ()))
        mq = lax.dot_general(wq, v8, dn, preferred_element_type=jnp.float32)
        bfv = lax.dot_general(v8, bf8_ref[...], dn,
                              preferred_element_type=jnp.float32)
        wa_eff = jnp.concatenate([wa_ref[0:_FLAG_ROW], mq], axis=0)
        bias = bcat_ref[...] + bfv
        gam = gam_ref[...]
        bet = bet_ref[...]
        for g in range(grp):
            tile = ct_ref[g * _TR:(g + 1) * _TR, :]
            big = lax.dot_general(wa_eff, tile, dn,
                                  preferred_element_type=jnp.float32)
            big = big + bias
            gate = jax.nn.sigmoid(big[:64, :])
            z = gate * big[64:, :]
            mu = jnp.mean(z, axis=0, keepdims=True)
            d = z - mu
            var = jnp.mean(d * d, axis=0, keepdims=True)
            o = d * lax.rsqrt(var + 1e-5) * gam + bet
            out_ref[0, :, pl.ds(g * 128, 128)] = o

    return pl.pallas_call(
        body,
        grid=(seq, bblk),
        in_specs=[
            pl.BlockSpec((grp * _TR, 128), lambda i, j: (i * bblk + j, 0)),
            pl.BlockSpec((_TR, 128), lambda i, j: (0, 0)),
            pl.BlockSpec((8, 128), lambda i, j: (0, 0)),
            pl.BlockSpec((8, 8), lambda i, j: (0, 0)),
            pl.BlockSpec((8, 1), lambda i, j: (0, 0)),
            pl.BlockSpec((128, 1), lambda i, j: (0, 0)),
            pl.BlockSpec((64, 1), lambda i, j: (0, 0)),
            pl.BlockSpec((64, 1), lambda i, j: (0, 0)),
        ],
        out_specs=pl.BlockSpec((1, 64, grp * 128), lambda i, j: (i, 0, j)),
        out_shape=jax.ShapeDtypeStruct((seq, 64, bsz), jnp.float32),
        compiler_params=pltpu.CompilerParams(
            dimension_semantics=("arbitrary", "arbitrary"),
            fuse_transposed_lhs_in_matmul=True),
        name="tc_fused_dense",
    )


def kernel(page_hash, offset, cache_line, addr_alignment, stride, reuse_dist,
           locality_cluster, entropy, address_flags, tbl_ph, tbl_off, tbl_cl,
           tbl_aa, tbl_str, tbl_rd, tbl_lc, tbl_ent, W_flags, b_flags,
           W_gate, b_gate, W_tr, b_tr, gamma, beta):
    bsz, seq = page_hash.shape
    n = bsz * seq
    nt = n // 128
    idx_t = [jnp.transpose(a).reshape(n).astype(jnp.int32)
             for a in (page_hash, offset, cache_line, addr_alignment, stride,
                       reuse_dist, locality_cluster, entropy)]
    idx3 = jnp.stack(idx_t, axis=0).reshape(8, nt, 128).transpose(1, 0, 2)
    flags_flat = jnp.transpose(address_flags, (1, 0, 2)).reshape(n * 5)
    tbls = [tbl_ph, tbl_off, tbl_cl, tbl_aa, tbl_str, tbl_rd, tbl_lc, tbl_ent]
    tbls = [t if t.shape[1] == 8 else jnp.pad(t, ((0, 0), (0, 8 - t.shape[1])))
            for t in tbls]
    comb2 = _sc_gather(n)(idx3, flags_flat, *tbls)

    wcat_t = jnp.concatenate([W_gate, W_tr], axis=0).T
    wa = jnp.zeros((_TR, 128), jnp.float32).at[:_NFEAT].set(wcat_t[:_NFEAT])
    v8 = jnp.zeros((8, 128), jnp.float32).at[:5].set(wcat_t[_NFEAT:44])
    wf8 = jnp.zeros((8, 8), jnp.float32).at[:5, :5].set(W_flags)
    bf8 = jnp.zeros((8, 1), jnp.float32).at[:5, 0].set(b_flags)
    bcat = jnp.concatenate([b_gate, b_tr]).reshape(128, 1)
    outp = _tc_dense(bsz, seq)(comb2, wa, v8, wf8, bf8, bcat,
                               gamma.reshape(64, 1), beta.reshape(64, 1))
    return jnp.transpose(outp, (2, 0, 1))


# speedup vs baseline: 14.9003x; 1.0471x over previous
import functools

import jax
import jax.numpy as jnp
from jax import lax
from jax.experimental import pallas as pl
from jax.experimental.pallas import tpu as pltpu
from jax.experimental.pallas import tpu_sc as plsc

_D = (8, 4, 4, 3, 6, 5, 6, 3)
_OFF = (0, 8, 12, 16, 19, 25, 30, 36)
_NFEAT = 39
_FLAG_ROW = 40
_TR = 48
_NC, _NS = 2, 16
_NW = _NC * _NS


class _Slot:
    def __init__(self, ib, fb, gb, tile, sem_l, sem_g, sem_s):
        self.ib, self.fb, self.gb, self.tile = ib, fb, gb, tile
        self.sem_l, self.sem_g, self.sem_s = sem_l, sem_g, sem_s


@functools.lru_cache(maxsize=2)
def _sc_gather(n_tokens):
    nt = n_tokens // 128
    tiles_per_w = nt // _NW
    assert tiles_per_w % 2 == 0 and tiles_per_w >= 4
    mesh = plsc.VectorSubcoreMesh(
        core_axis_name="c", subcore_axis_name="s",
        num_cores=_NC, num_subcores=_NS)

    def body(i0, i1, i2, i3, i4, i5, i6, i7,
             flags_ref, t0, t1, t2, t3, t4, t5, t6, t7, out_ref,
             ib_a, fb_a, ga0, ga1, ga2, ga3, ga4, ga5, ga6, ga7, tile_a,
             ib_b, fb_b, gb0, gb1, gb2, gb3, gb4, gb5, gb6, gb7, tile_b,
             sem_la, sem_ga, sem_sa, sem_lb, sem_gb, sem_sb):
        idx_refs = (i0, i1, i2, i3, i4, i5, i6, i7)
        tbl_refs = (t0, t1, t2, t3, t4, t5, t6, t7)
        sa = _Slot(ib_a, fb_a, (ga0, ga1, ga2, ga3, ga4, ga5, ga6, ga7),
                   tile_a, sem_la, sem_ga, sem_sa)
        sb = _Slot(ib_b, fb_b, (gb0, gb1, gb2, gb3, gb4, gb5, gb6, gb7),
                   tile_b, sem_lb, sem_gb, sem_sb)

        wid = lax.axis_index("s") * _NC + lax.axis_index("c")
        wbase = wid * tiles_per_w
        iota = lax.iota(jnp.int32, 16)
        zeros = jnp.zeros((16,), jnp.float32)
        cols = [jnp.full((16,), k, jnp.int32) for k in range(8)]

        def fire_load(s, gt):
            for i in range(8):
                pltpu.async_copy(
                    idx_refs[i].at[pl.ds(gt * 128, 128)], s.ib.at[i], s.sem_l)
            pltpu.async_copy(flags_ref.at[pl.ds(gt * 640, 640)], s.fb, s.sem_l)

        def wait_load(s, gt):
            for i in range(8):
                pltpu.make_async_copy(
                    idx_refs[i].at[pl.ds(gt * 128, 128)], s.ib.at[i],
                    s.sem_l).wait()
            pltpu.make_async_copy(
                flags_ref.at[pl.ds(gt * 640, 640)], s.fb, s.sem_l).wait()

        def fire_gather(s):
            for i in range(8):
                pltpu.async_copy(tbl_refs[i].at[s.ib.at[i]], s.gb[i], s.sem_g)

        def wait_gather(s):
            for i in range(8):
                pltpu.make_async_copy(
                    tbl_refs[i].at[s.ib.at[i]], s.gb[i], s.sem_g).wait()

        def transpose_flags(s):
            for lg in range(8):
                fbase = iota * 5 + lg * 80
                for k in range(5):
                    v = plsc.load_gather(s.fb, [fbase + k])
                    s.tile[_FLAG_ROW + k, pl.ds(lg * 16, 16)] = v

        def transpose_emb(s):
            for lg in range(8):
                rows = iota + lg * 16
                for i in range(8):
                    for k in range(_D[i]):
                        v = plsc.load_gather(s.gb[i], [rows, cols[k]])
                        s.tile[_OFF[i] + k, pl.ds(lg * 16, 16)] = v

        def fire_store(s, gt):
            pltpu.async_copy(s.tile, out_ref.at[pl.ds(gt * _TR, _TR), :],
                             s.sem_s)

        def wait_store(s, gt_old):
            pltpu.make_async_copy(
                s.tile, out_ref.at[pl.ds(gt_old * _TR, _TR), :], s.sem_s
            ).wait()

        for s in (sa, sb):
            for r in (39, 45, 46, 47):
                for lg in range(8):
                    s.tile[r, pl.ds(lg * 16, 16)] = zeros

        fire_load(sa, wbase)
        wait_load(sa, wbase)
        fire_gather(sa)
        fire_load(sb, wbase + 1)

        def half(k, s, o, t_cur, t_nxt):
            wait_load(o, t_nxt)
            fire_gather(o)
            wait_gather(s)

            @pl.when(k > 0)
            def _():
                wait_store(s, t_cur - 2)

            transpose_flags(s)
            fire_load(s, t_nxt + 1)
            transpose_emb(s)
            fire_store(s, t_cur)

        def step(k, carry):
            t = wbase + 2 * k
            half(k, sa, sb, t, t + 1)
            half(k, sb, sa, t + 1, t + 2)
            return carry

        lax.fori_loop(0, tiles_per_w // 2 - 1, step, 0)

        t_last = wbase + tiles_per_w - 1
        wait_gather(sa)
        wait_store(sa, t_last - 3)
        transpose_flags(sa)
        transpose_emb(sa)
        fire_store(sa, t_last - 1)
        wait_load(sb, t_last)
        fire_gather(sb)
        wait_gather(sb)
        wait_store(sb, t_last - 2)
        transpose_flags(sb)
        transpose_emb(sb)
        fire_store(sb, t_last)
        wait_store(sa, t_last - 1)
        wait_store(sb, t_last)

    scratch = (
        [pltpu.VMEM((8, 128), jnp.int32), pltpu.VMEM((640,), jnp.float32)]
        + [pltpu.VMEM((128, 8), jnp.float32) for _ in range(8)]
        + [pltpu.VMEM((_TR, 128), jnp.float32)]
        + [pltpu.VMEM((8, 128), jnp.int32), pltpu.VMEM((640,), jnp.float32)]
        + [pltpu.VMEM((128, 8), jnp.float32) for _ in range(8)]
        + [pltpu.VMEM((_TR, 128), jnp.float32)]
        + [pltpu.SemaphoreType.DMA] * 6
    )
    return pl.kernel(
        body,
        out_type=jax.ShapeDtypeStruct((nt * _TR, 128), jnp.float32),
        mesh=mesh,
        scratch_types=scratch,
        compiler_params=pltpu.CompilerParams(
            needs_layout_passes=False, use_tc_tiling_on_sc=False),
        name="sc_embed_gather",
    )


@functools.lru_cache(maxsize=2)
def _tc_dense(bsz, seq):
    grp = 8
    bblk = bsz // (grp * 128)

    def body(ct_ref, wa_ref, v8_ref, wf8_ref, bf8_ref, bcat_ref,
             gam_ref, bet_ref, out_ref):
        wf = wf8_ref[...]
        scale = jnp.sum(jnp.abs(wf)) / 25.0 + 1e-8
        wq = jnp.clip(jnp.round(wf / scale), -1.0, 1.0) * scale
        v8 = v8_ref[...]
        dn = (((0,), (0,)), ((), ()))
        mq = lax.dot_general(wq, v8, dn, preferred_element_type=jnp.float32)
        bfv = lax.dot_general(v8, bf8_ref[...], dn,
                              preferred_element_type=jnp.float32)
        wa_eff = jnp.concatenate([wa_ref[0:_FLAG_ROW], mq], axis=0)
        bias = bcat_ref[...] + bfv
        gam = gam_ref[...]
        bet = bet_ref[...]
        for g in range(grp):
            tile = ct_ref[g * _TR:(g + 1) * _TR, :]
            big = lax.dot_general(wa_eff, tile, dn,
                                  preferred_element_type=jnp.float32)
            big = big + bias
            gate = jax.nn.sigmoid(big[:64, :])
            z = gate * big[64:, :]
            mu = jnp.mean(z, axis=0, keepdims=True)
            d = z - mu
            var = jnp.mean(d * d, axis=0, keepdims=True)
            o = d * lax.rsqrt(var + 1e-5) * gam + bet
            out_ref[0, :, pl.ds(g * 128, 128)] = o

    return pl.pallas_call(
        body,
        grid=(seq, bblk),
        in_specs=[
            pl.BlockSpec((grp * _TR, 128), lambda i, j: (i * bblk + j, 0)),
            pl.BlockSpec((_TR, 128), lambda i, j: (0, 0)),
            pl.BlockSpec((8, 128), lambda i, j: (0, 0)),
            pl.BlockSpec((8, 8), lambda i, j: (0, 0)),
            pl.BlockSpec((8, 1), lambda i, j: (0, 0)),
            pl.BlockSpec((128, 1), lambda i, j: (0, 0)),
            pl.BlockSpec((64, 1), lambda i, j: (0, 0)),
            pl.BlockSpec((64, 1), lambda i, j: (0, 0)),
        ],
        out_specs=pl.BlockSpec((1, 64, grp * 128), lambda i, j: (i, 0, j)),
        out_shape=jax.ShapeDtypeStruct((seq, 64, bsz), jnp.float32),
        compiler_params=pltpu.CompilerParams(
            dimension_semantics=("arbitrary", "arbitrary"),
            fuse_transposed_lhs_in_matmul=True),
        name="tc_fused_dense",
    )


def kernel(page_hash, offset, cache_line, addr_alignment, stride, reuse_dist,
           locality_cluster, entropy, address_flags, tbl_ph, tbl_off, tbl_cl,
           tbl_aa, tbl_str, tbl_rd, tbl_lc, tbl_ent, W_flags, b_flags,
           W_gate, b_gate, W_tr, b_tr, gamma, beta):
    bsz, seq = page_hash.shape
    n = bsz * seq
    nt = n // 128
    idx_t = [jnp.transpose(a).reshape(n).astype(jnp.int32)
             for a in (page_hash, offset, cache_line, addr_alignment, stride,
                       reuse_dist, locality_cluster, entropy)]
    flags_flat = jnp.transpose(address_flags, (1, 0, 2)).reshape(n * 5)
    tbls = [tbl_ph, tbl_off, tbl_cl, tbl_aa, tbl_str, tbl_rd, tbl_lc, tbl_ent]
    tbls = [t if t.shape[1] == 8 else jnp.pad(t, ((0, 0), (0, 8 - t.shape[1])))
            for t in tbls]
    comb2 = _sc_gather(n)(*idx_t, flags_flat, *tbls)

    wcat_t = jnp.concatenate([W_gate, W_tr], axis=0).T
    wa = jnp.zeros((_TR, 128), jnp.float32).at[:_NFEAT].set(wcat_t[:_NFEAT])
    v8 = jnp.zeros((8, 128), jnp.float32).at[:5].set(wcat_t[_NFEAT:44])
    wf8 = jnp.zeros((8, 8), jnp.float32).at[:5, :5].set(W_flags)
    bf8 = jnp.zeros((8, 1), jnp.float32).at[:5, 0].set(b_flags)
    bcat = jnp.concatenate([b_gate, b_tr]).reshape(128, 1)
    outp = _tc_dense(bsz, seq)(comb2, wa, v8, wf8, bf8, bcat,
                               gamma.reshape(64, 1), beta.reshape(64, 1))
    return jnp.transpose(outp, (2, 0, 1))


# speedup vs baseline: 18.6891x; 1.2543x over previous
import functools

import jax
import jax.numpy as jnp
from jax import lax
from jax.experimental import pallas as pl
from jax.experimental.pallas import tpu as pltpu
from jax.experimental.pallas import tpu_sc as plsc

_D = (8, 4, 4, 3, 6, 5, 6, 3)
_OFF = (0, 8, 12, 16, 19, 25, 30, 36)
_NFEAT = 39
_FLAG_ROW = 40
_TR = 48
_NC, _NS = 2, 16
_NW = _NC * _NS


class _Slot:
    def __init__(self, ib, gb, tile, sem_l, sem_g, sem_s, sem_f):
        self.ib, self.gb, self.tile = ib, gb, tile
        self.sem_l, self.sem_g, self.sem_s, self.sem_f = (
            sem_l, sem_g, sem_s, sem_f)


@functools.lru_cache(maxsize=2)
def _sc_gather(n_tokens, bsz):
    nt = n_tokens // 128
    tiles_per_w = nt // _NW
    tpl = bsz // 128
    assert tiles_per_w % 2 == 0 and tiles_per_w >= 4
    mesh = plsc.VectorSubcoreMesh(
        core_axis_name="c", subcore_axis_name="s",
        num_cores=_NC, num_subcores=_NS)

    def body(i0, i1, i2, i3, i4, i5, i6, i7, flags_ref,
             t0, t1, t2, t3, t4, t5, t6, t7, out_ref,
             ib_a, ga0, ga1, ga2, ga3, ga4, ga5, ga6, ga7, tile_a,
             ib_b, gb0, gb1, gb2, gb3, gb4, gb5, gb6, gb7, tile_b,
             sem_la, sem_ga, sem_sa, sem_fa, sem_lb, sem_gb, sem_sb, sem_fb):
        idx_refs = (i0, i1, i2, i3, i4, i5, i6, i7)
        tbl_refs = (t0, t1, t2, t3, t4, t5, t6, t7)
        sa = _Slot(ib_a, (ga0, ga1, ga2, ga3, ga4, ga5, ga6, ga7),
                   tile_a, sem_la, sem_ga, sem_sa, sem_fa)
        sb = _Slot(ib_b, (gb0, gb1, gb2, gb3, gb4, gb5, gb6, gb7),
                   tile_b, sem_lb, sem_gb, sem_sb, sem_fb)

        wid = lax.axis_index("s") * _NC + lax.axis_index("c")
        wbase = wid * tiles_per_w
        iota = lax.iota(jnp.int32, 16)
        zeros = jnp.zeros((16,), jnp.float32)
        cols = [jnp.full((16,), k, jnp.int32) for k in range(8)]

        def _flags_src(gt):
            l = gt // tpl
            b0 = (gt % tpl) * 128
            return flags_ref.at[l, :, pl.ds(b0, 128)]

        def fire_load(s, gt):
            for i in range(8):
                pltpu.async_copy(
                    idx_refs[i].at[pl.ds(gt * 128, 128)], s.ib.at[i], s.sem_l)

        def wait_load(s, gt):
            for i in range(8):
                pltpu.make_async_copy(
                    idx_refs[i].at[pl.ds(gt * 128, 128)], s.ib.at[i],
                    s.sem_l).wait()

        def fire_flags(s, gt):
            pltpu.async_copy(
                _flags_src(gt), s.tile.at[pl.ds(_FLAG_ROW, 5), :], s.sem_f)

        def wait_flags(s, gt):
            pltpu.make_async_copy(
                _flags_src(gt), s.tile.at[pl.ds(_FLAG_ROW, 5), :],
                s.sem_f).wait()

        def fire_gather(s):
            for i in range(8):
                pltpu.async_copy(tbl_refs[i].at[s.ib.at[i]], s.gb[i], s.sem_g)

        def wait_gather(s):
            for i in range(8):
                pltpu.make_async_copy(
                    tbl_refs[i].at[s.ib.at[i]], s.gb[i], s.sem_g).wait()

        def transpose_emb(s):
            for lg in range(8):
                rows = iota + lg * 16
                for i in range(8):
                    for k in range(_D[i]):
                        v = plsc.load_gather(s.gb[i], [rows, cols[k]])
                        s.tile[_OFF[i] + k, pl.ds(lg * 16, 16)] = v

        def fire_store(s, gt):
            pltpu.async_copy(s.tile, out_ref.at[pl.ds(gt * _TR, _TR), :],
                             s.sem_s)

        def wait_store(s, gt_old):
            pltpu.make_async_copy(
                s.tile, out_ref.at[pl.ds(gt_old * _TR, _TR), :], s.sem_s
            ).wait()

        for s in (sa, sb):
            for r in (39, 45, 46, 47):
                for lg in range(8):
                    s.tile[r, pl.ds(lg * 16, 16)] = zeros

        fire_load(sa, wbase)
        wait_load(sa, wbase)
        fire_gather(sa)
        fire_load(sb, wbase + 1)

        def half(k, s, o, t_cur, t_nxt):
            wait_load(o, t_nxt)
            fire_gather(o)
            wait_gather(s)

            @pl.when(k > 0)
            def _():
                wait_store(s, t_cur - 2)

            fire_flags(s, t_cur)
            fire_load(s, t_nxt + 1)
            transpose_emb(s)
            wait_flags(s, t_cur)
            fire_store(s, t_cur)

        def step(k, carry):
            t = wbase + 2 * k
            half(k, sa, sb, t, t + 1)
            half(k, sb, sa, t + 1, t + 2)
            return carry

        lax.fori_loop(0, tiles_per_w // 2 - 1, step, 0)

        t_last = wbase + tiles_per_w - 1
        wait_gather(sa)
        wait_store(sa, t_last - 3)
        fire_flags(sa, t_last - 1)
        transpose_emb(sa)
        wait_flags(sa, t_last - 1)
        fire_store(sa, t_last - 1)
        wait_load(sb, t_last)
        fire_gather(sb)
        wait_gather(sb)
        wait_store(sb, t_last - 2)
        fire_flags(sb, t_last)
        transpose_emb(sb)
        wait_flags(sb, t_last)
        fire_store(sb, t_last)
        wait_store(sa, t_last - 1)
        wait_store(sb, t_last)

    scratch = (
        [pltpu.VMEM((8, 128), jnp.int32)]
        + [pltpu.VMEM((128, 8), jnp.float32) for _ in range(8)]
        + [pltpu.VMEM((_TR, 128), jnp.float32)]
        + [pltpu.VMEM((8, 128), jnp.int32)]
        + [pltpu.VMEM((128, 8), jnp.float32) for _ in range(8)]
        + [pltpu.VMEM((_TR, 128), jnp.float32)]
        + [pltpu.SemaphoreType.DMA] * 8
    )
    return pl.kernel(
        body,
        out_type=jax.ShapeDtypeStruct((nt * _TR, 128), jnp.float32),
        mesh=mesh,
        scratch_types=scratch,
        compiler_params=pltpu.CompilerParams(
            needs_layout_passes=False, use_tc_tiling_on_sc=False),
        name="sc_embed_gather",
    )


@functools.lru_cache(maxsize=2)
def _tc_dense(bsz, seq):
    grp = 8
    bblk = bsz // (grp * 128)

    def body(ct_ref, wa_ref, v8_ref, wf8_ref, bf8_ref, bcat_ref,
             gam_ref, bet_ref, out_ref):
        wf = wf8_ref[...]
        scale = jnp.sum(jnp.abs(wf)) / 25.0 + 1e-8
        wq = jnp.clip(jnp.round(wf / scale), -1.0, 1.0) * scale
        v8 = v8_ref[...]
        dn = (((0,), (0,)), ((), ()))
        mq = lax.dot_general(wq, v8, dn, preferred_element_type=jnp.float32)
        bfv = lax.dot_general(v8, bf8_ref[...], dn,
                              preferred_element_type=jnp.float32)
        bias = bcat_ref[...] + bfv
        gam = gam_ref[...]
        bet = bet_ref[...]
        wa_eff = jnp.concatenate([wa_ref[0:_FLAG_ROW], mq], axis=0)
        for g in range(grp):
            tile = ct_ref[g * _TR:(g + 1) * _TR, :]
            big = lax.dot_general(wa_eff, tile, dn,
                                  preferred_element_type=jnp.float32)
            big = big + bias
            gate = jax.nn.sigmoid(big[:64, :])
            z = gate * big[64:, :]
            mu = jnp.mean(z, axis=0, keepdims=True)
            d = z - mu
            var = jnp.mean(d * d, axis=0, keepdims=True)
            o = d * lax.rsqrt(var + 1e-5) * gam + bet
            out_ref[0, :, pl.ds(g * 128, 128)] = o

    return pl.pallas_call(
        body,
        grid=(seq, bblk),
        in_specs=[
            pl.BlockSpec((grp * _TR, 128), lambda i, j: (i * bblk + j, 0)),
            pl.BlockSpec((_TR, 128), lambda i, j: (0, 0)),
            pl.BlockSpec((8, 128), lambda i, j: (0, 0)),
            pl.BlockSpec((8, 8), lambda i, j: (0, 0)),
            pl.BlockSpec((8, 1), lambda i, j: (0, 0)),
            pl.BlockSpec((128, 1), lambda i, j: (0, 0)),
            pl.BlockSpec((64, 1), lambda i, j: (0, 0)),
            pl.BlockSpec((64, 1), lambda i, j: (0, 0)),
        ],
        out_specs=pl.BlockSpec((1, 64, grp * 128), lambda i, j: (i, 0, j)),
        out_shape=jax.ShapeDtypeStruct((seq, 64, bsz), jnp.float32),
        compiler_params=pltpu.CompilerParams(
            dimension_semantics=("arbitrary", "arbitrary"),
            fuse_transposed_lhs_in_matmul=True),
        name="tc_fused_dense",
    )


def kernel(page_hash, offset, cache_line, addr_alignment, stride, reuse_dist,
           locality_cluster, entropy, address_flags, tbl_ph, tbl_off, tbl_cl,
           tbl_aa, tbl_str, tbl_rd, tbl_lc, tbl_ent, W_flags, b_flags,
           W_gate, b_gate, W_tr, b_tr, gamma, beta):
    bsz, seq = page_hash.shape
    n = bsz * seq
    nt = n // 128
    idx_t = [jnp.transpose(a).reshape(n).astype(jnp.int32)
             for a in (page_hash, offset, cache_line, addr_alignment, stride,
                       reuse_dist, locality_cluster, entropy)]
    tbls = [tbl_ph, tbl_off, tbl_cl, tbl_aa, tbl_str, tbl_rd, tbl_lc, tbl_ent]
    tbls = [t if t.shape[1] == 8 else jnp.pad(t, ((0, 0), (0, 8 - t.shape[1])))
            for t in tbls]
    flags_t = jnp.transpose(address_flags, (1, 2, 0))
    comb2 = _sc_gather(n, bsz)(*idx_t, flags_t, *tbls)

    wcat_t = jnp.concatenate([W_gate, W_tr], axis=0).T
    wa = jnp.zeros((_TR, 128), jnp.float32).at[:_NFEAT].set(wcat_t[:_NFEAT])
    v8 = jnp.zeros((8, 128), jnp.float32).at[:5].set(wcat_t[_NFEAT:44])
    wf8 = jnp.zeros((8, 8), jnp.float32).at[:5, :5].set(W_flags)
    bf8 = jnp.zeros((8, 1), jnp.float32).at[:5, 0].set(b_flags)
    bcat = jnp.concatenate([b_gate, b_tr]).reshape(128, 1)
    outp = _tc_dense(bsz, seq)(comb2, wa, v8, wf8, bf8, bcat,
                               gamma.reshape(64, 1), beta.reshape(64, 1))
    return jnp.transpose(outp, (2, 0, 1))


# speedup vs baseline: 23.5359x; 1.2593x over previous
import functools

import jax
import jax.numpy as jnp
from jax import lax
from jax.experimental import pallas as pl
from jax.experimental.pallas import tpu as pltpu
from jax.experimental.pallas import tpu_sc as plsc

_D = (8, 4, 4, 3, 6, 5, 6, 3)
_OFF = (0, 8, 12, 16, 19, 25, 30, 36)
_NFEAT = 39
_FLAG_ROW = 40
_TR = 48
_NC, _NS = 2, 16
_NW = _NC * _NS


class _Slot:
    def __init__(self, ib, gb, tile, sem_l, sem_g, sem_s, sem_f):
        self.ib, self.gb, self.tile = ib, gb, tile
        self.sem_l, self.sem_g, self.sem_s, self.sem_f = (
            sem_l, sem_g, sem_s, sem_f)


@functools.lru_cache(maxsize=2)
def _sc_gather(n_tokens, bsz):
    nt = n_tokens // 128
    tiles_per_w = nt // _NW
    tpl = bsz // 128
    assert tiles_per_w % 2 == 0 and tiles_per_w >= 4
    mesh = plsc.VectorSubcoreMesh(
        core_axis_name="c", subcore_axis_name="s",
        num_cores=_NC, num_subcores=_NS)

    def body(i0, i1, i2, i3, i4, i5, i6, i7, flags_ref,
             t0, t1, t2, t3, t4, t5, t6, t7, out_ref,
             ib_a, ga0, ga1, ga2, ga3, ga4, ga5, ga6, ga7, tile_a,
             ib_b, gb0, gb1, gb2, gb3, gb4, gb5, gb6, gb7, tile_b,
             sem_la, sem_ga, sem_sa, sem_fa, sem_lb, sem_gb, sem_sb, sem_fb):
        idx_refs = (i0, i1, i2, i3, i4, i5, i6, i7)
        tbl_refs = (t0, t1, t2, t3, t4, t5, t6, t7)
        sa = _Slot(ib_a, (ga0, ga1, ga2, ga3, ga4, ga5, ga6, ga7),
                   tile_a, sem_la, sem_ga, sem_sa, sem_fa)
        sb = _Slot(ib_b, (gb0, gb1, gb2, gb3, gb4, gb5, gb6, gb7),
                   tile_b, sem_lb, sem_gb, sem_sb, sem_fb)

        wid = lax.axis_index("s") * _NC + lax.axis_index("c")
        wbase = wid * tiles_per_w
        iota = lax.iota(jnp.int32, 16)
        zeros = jnp.zeros((16,), jnp.float32)
        cols = [jnp.full((16,), k, jnp.int32) for k in range(8)]

        def _flags_src(gt):
            l = gt // tpl
            b0 = (gt % tpl) * 128
            return flags_ref.at[l, :, pl.ds(b0, 128)]

        def fire_load(s, gt):
            for i in range(8):
                pltpu.async_copy(
                    idx_refs[i].at[pl.ds(gt * 128, 128)], s.ib.at[i], s.sem_l)

        def wait_load(s, gt):
            for i in range(8):
                pltpu.make_async_copy(
                    idx_refs[i].at[pl.ds(gt * 128, 128)], s.ib.at[i],
                    s.sem_l).wait()

        def fire_flags(s, gt):
            pltpu.async_copy(
                _flags_src(gt), s.tile.at[pl.ds(_FLAG_ROW, 5), :], s.sem_f)

        def wait_flags(s, gt):
            pltpu.make_async_copy(
                _flags_src(gt), s.tile.at[pl.ds(_FLAG_ROW, 5), :],
                s.sem_f).wait()

        def fire_gather(s):
            for i in range(8):
                pltpu.async_copy(tbl_refs[i].at[s.ib.at[i]], s.gb[i], s.sem_g)

        def wait_gather(s):
            for i in range(8):
                pltpu.make_async_copy(
                    tbl_refs[i].at[s.ib.at[i]], s.gb[i], s.sem_g).wait()

        def transpose_emb(s):
            for lg in range(8):
                rows = iota + lg * 16
                for i in range(8):
                    for k in range(_D[i]):
                        v = plsc.load_gather(s.gb[i], [rows, cols[k]])
                        s.tile[_OFF[i] + k, pl.ds(lg * 16, 16)] = v

        def fire_store(s, gt):
            pltpu.async_copy(s.tile, out_ref.at[pl.ds(gt * _TR, _TR), :],
                             s.sem_s)

        def wait_store(s, gt_old):
            pltpu.make_async_copy(
                s.tile, out_ref.at[pl.ds(gt_old * _TR, _TR), :], s.sem_s
            ).wait()

        for s in (sa, sb):
            for r in (39, 45, 46, 47):
                for lg in range(8):
                    s.tile[r, pl.ds(lg * 16, 16)] = zeros

        fire_load(sa, wbase)
        wait_load(sa, wbase)
        fire_gather(sa)
        fire_load(sb, wbase + 1)

        def half(k, s, o, t_cur, t_nxt):
            wait_load(o, t_nxt)
            fire_gather(o)
            wait_gather(s)

            @pl.when(k > 0)
            def _():
                wait_store(s, t_cur - 2)

            fire_flags(s, t_cur)
            fire_load(s, t_nxt + 1)
            transpose_emb(s)
            wait_flags(s, t_cur)
            fire_store(s, t_cur)

        def step(k, carry):
            t = wbase + 2 * k
            half(k, sa, sb, t, t + 1)
            half(k, sb, sa, t + 1, t + 2)
            return carry

        lax.fori_loop(0, tiles_per_w // 2 - 1, step, 0)

        t_last = wbase + tiles_per_w - 1
        wait_gather(sa)
        wait_store(sa, t_last - 3)
        fire_flags(sa, t_last - 1)
        transpose_emb(sa)
        wait_flags(sa, t_last - 1)
        fire_store(sa, t_last - 1)
        wait_load(sb, t_last)
        fire_gather(sb)
        wait_gather(sb)
        wait_store(sb, t_last - 2)
        fire_flags(sb, t_last)
        transpose_emb(sb)
        wait_flags(sb, t_last)
        fire_store(sb, t_last)
        wait_store(sa, t_last - 1)
        wait_store(sb, t_last)

    scratch = (
        [pltpu.VMEM((8, 128), jnp.int32)]
        + [pltpu.VMEM((128, 8), jnp.float32) for _ in range(8)]
        + [pltpu.VMEM((_TR, 128), jnp.float32)]
        + [pltpu.VMEM((8, 128), jnp.int32)]
        + [pltpu.VMEM((128, 8), jnp.float32) for _ in range(8)]
        + [pltpu.VMEM((_TR, 128), jnp.float32)]
        + [pltpu.SemaphoreType.DMA] * 8
    )
    return pl.kernel(
        body,
        out_type=jax.ShapeDtypeStruct((nt * _TR, 128), jnp.float32),
        mesh=mesh,
        scratch_types=scratch,
        compiler_params=pltpu.CompilerParams(
            needs_layout_passes=False, use_tc_tiling_on_sc=False),
        name="sc_embed_gather",
    )


@functools.lru_cache(maxsize=2)
def _tc_dense(bsz, seq):
    grp = 16
    bblk = bsz // (grp * 128)

    def body(ct_ref, wa_ref, v8_ref, wf8_ref, bf8_ref, bcat_ref,
             gam_ref, bet_ref, out_ref):
        wf = wf8_ref[...]
        scale = jnp.sum(jnp.abs(wf)) / 25.0 + 1e-8
        wq = jnp.clip(jnp.round(wf / scale), -1.0, 1.0) * scale
        v8 = v8_ref[...]
        dn = (((0,), (0,)), ((), ()))
        mq = lax.dot_general(wq, v8, dn, preferred_element_type=jnp.float32)
        bfv = lax.dot_general(v8, bf8_ref[...], dn,
                              preferred_element_type=jnp.float32)
        bias = bcat_ref[...] + bfv
        gam = gam_ref[...]
        bet = bet_ref[...]
        wa_eff = jnp.concatenate([wa_ref[0:_FLAG_ROW], mq], axis=0)
        for g in range(grp):
            tile = ct_ref[g * _TR:(g + 1) * _TR, :]
            big = lax.dot_general(wa_eff, tile, dn,
                                  preferred_element_type=jnp.float32)
            big = big + bias
            gate = jax.nn.sigmoid(big[:64, :])
            z = gate * big[64:, :]
            mu = jnp.mean(z, axis=0, keepdims=True)
            d = z - mu
            var = jnp.mean(d * d, axis=0, keepdims=True)
            o = d * lax.rsqrt(var + 1e-5) * gam + bet
            out_ref[0, :, pl.ds(g * 128, 128)] = o

    return pl.pallas_call(
        body,
        grid=(seq, bblk),
        in_specs=[
            pl.BlockSpec((grp * _TR, 128), lambda i, j: (i * bblk + j, 0)),
            pl.BlockSpec((_TR, 128), lambda i, j: (0, 0)),
            pl.BlockSpec((8, 128), lambda i, j: (0, 0)),
            pl.BlockSpec((8, 8), lambda i, j: (0, 0)),
            pl.BlockSpec((8, 1), lambda i, j: (0, 0)),
            pl.BlockSpec((128, 1), lambda i, j: (0, 0)),
            pl.BlockSpec((64, 1), lambda i, j: (0, 0)),
            pl.BlockSpec((64, 1), lambda i, j: (0, 0)),
        ],
        out_specs=pl.BlockSpec((1, 64, grp * 128), lambda i, j: (i, 0, j)),
        out_shape=jax.ShapeDtypeStruct((seq, 64, bsz), jnp.float32),
        compiler_params=pltpu.CompilerParams(
            dimension_semantics=("arbitrary", "arbitrary"),
            fuse_transposed_lhs_in_matmul=True),
        name="tc_fused_dense",
    )


def kernel(page_hash, offset, cache_line, addr_alignment, stride, reuse_dist,
           locality_cluster, entropy, address_flags, tbl_ph, tbl_off, tbl_cl,
           tbl_aa, tbl_str, tbl_rd, tbl_lc, tbl_ent, W_flags, b_flags,
           W_gate, b_gate, W_tr, b_tr, gamma, beta):
    bsz, seq = page_hash.shape
    n = bsz * seq
    nt = n // 128
    idx_t = [jnp.transpose(a).reshape(n).astype(jnp.int32)
             for a in (page_hash, offset, cache_line, addr_alignment, stride,
                       reuse_dist, locality_cluster, entropy)]
    tbls = [tbl_ph, tbl_off, tbl_cl, tbl_aa, tbl_str, tbl_rd, tbl_lc, tbl_ent]
    tbls = [t if t.shape[1] == 8 else jnp.pad(t, ((0, 0), (0, 8 - t.shape[1])))
            for t in tbls]
    flags_t = jnp.transpose(address_flags, (1, 2, 0))
    comb2 = _sc_gather(n, bsz)(*idx_t, flags_t, *tbls)

    wcat_t = jnp.concatenate([W_gate, W_tr], axis=0).T
    wa = jnp.zeros((_TR, 128), jnp.float32).at[:_NFEAT].set(wcat_t[:_NFEAT])
    v8 = jnp.zeros((8, 128), jnp.float32).at[:5].set(wcat_t[_NFEAT:44])
    wf8 = jnp.zeros((8, 8), jnp.float32).at[:5, :5].set(W_flags)
    bf8 = jnp.zeros((8, 1), jnp.float32).at[:5, 0].set(b_flags)
    bcat = jnp.concatenate([b_gate, b_tr]).reshape(128, 1)
    outp = _tc_dense(bsz, seq)(comb2, wa, v8, wf8, bf8, bcat,
                               gamma.reshape(64, 1), beta.reshape(64, 1))
    return jnp.transpose(outp, (2, 0, 1))


# speedup vs baseline: 26.8543x; 1.1410x over previous
import functools

import jax
import jax.numpy as jnp
from jax import lax
from jax.experimental import pallas as pl
from jax.experimental.pallas import tpu as pltpu
from jax.experimental.pallas import tpu_sc as plsc

_D = (8, 4, 4, 3, 6, 5, 6, 3)
_OFF = (0, 8, 12, 16, 19, 25, 30, 36)
_NFEAT = 39
_FLAG_ROW = 40
_TR = 48
_NC, _NS = 2, 16
_NW = _NC * _NS


class _Slot:
    def __init__(self, ib, gb, tile, sem_l, sem_g, sem_s, sem_f):
        self.ib, self.gb, self.tile = ib, gb, tile
        self.sem_l, self.sem_g, self.sem_s, self.sem_f = (
            sem_l, sem_g, sem_s, sem_f)


@functools.lru_cache(maxsize=2)
def _sc_gather(n_tokens, bsz):
    nt = n_tokens // 128
    tiles_per_w = nt // _NW
    tpl = bsz // 128
    assert tiles_per_w % 2 == 0 and tiles_per_w >= 4
    mesh = plsc.VectorSubcoreMesh(
        core_axis_name="c", subcore_axis_name="s",
        num_cores=_NC, num_subcores=_NS)

    def body(i0, i1, i2, i3, i4, i5, i6, i7, flags_ref,
             t0, t1, t2, t3, t4, t5, t6, t7, out_ref,
             ib_a, ga0, ga1, ga2, ga3, ga4, ga5, ga6, ga7, tile_a,
             ib_b, gb0, gb1, gb2, gb3, gb4, gb5, gb6, gb7, tile_b,
             sem_la, sem_ga, sem_sa, sem_fa, sem_lb, sem_gb, sem_sb, sem_fb):
        idx_refs = (i0, i1, i2, i3, i4, i5, i6, i7)
        tbl_refs = (t0, t1, t2, t3, t4, t5, t6, t7)
        sa = _Slot(ib_a, (ga0, ga1, ga2, ga3, ga4, ga5, ga6, ga7),
                   tile_a, sem_la, sem_ga, sem_sa, sem_fa)
        sb = _Slot(ib_b, (gb0, gb1, gb2, gb3, gb4, gb5, gb6, gb7),
                   tile_b, sem_lb, sem_gb, sem_sb, sem_fb)

        wid = lax.axis_index("s") * _NC + lax.axis_index("c")
        wbase = wid * tiles_per_w
        iota = lax.iota(jnp.int32, 16)
        zeros = jnp.zeros((16,), jnp.float32)
        cols = [jnp.full((16,), k, jnp.int32) for k in range(8)]

        def _flags_src(gt):
            l = gt // tpl
            b0 = (gt % tpl) * 128
            return flags_ref.at[l, :, pl.ds(b0, 128)]

        def fire_load(s, gt):
            for i in range(8):
                pltpu.async_copy(
                    idx_refs[i].at[pl.ds(gt * 128, 128)], s.ib.at[i], s.sem_l)

        def wait_load(s, gt):
            for i in range(8):
                pltpu.make_async_copy(
                    idx_refs[i].at[pl.ds(gt * 128, 128)], s.ib.at[i],
                    s.sem_l).wait()

        def fire_flags(s, gt):
            pltpu.async_copy(
                _flags_src(gt), s.tile.at[pl.ds(_FLAG_ROW, 5), :], s.sem_f)

        def wait_flags(s, gt):
            pltpu.make_async_copy(
                _flags_src(gt), s.tile.at[pl.ds(_FLAG_ROW, 5), :],
                s.sem_f).wait()

        def fire_gather(s):
            for i in range(8):
                pltpu.async_copy(tbl_refs[i].at[s.ib.at[i]], s.gb[i], s.sem_g)

        def wait_gather(s):
            for i in range(8):
                pltpu.make_async_copy(
                    tbl_refs[i].at[s.ib.at[i]], s.gb[i], s.sem_g).wait()

        def transpose_emb(s):
            for lg in range(8):
                rows = iota + lg * 16
                for i in range(8):
                    for k in range(_D[i]):
                        v = plsc.load_gather(s.gb[i], [rows, cols[k]])
                        s.tile[_OFF[i] + k, pl.ds(lg * 16, 16)] = v

        def fire_store(s, gt):
            pltpu.async_copy(s.tile, out_ref.at[pl.ds(gt * _TR, _TR), :],
                             s.sem_s)

        def wait_store(s, gt_old):
            pltpu.make_async_copy(
                s.tile, out_ref.at[pl.ds(gt_old * _TR, _TR), :], s.sem_s
            ).wait()

        for s in (sa, sb):
            for r in (39, 45, 46, 47):
                for lg in range(8):
                    s.tile[r, pl.ds(lg * 16, 16)] = zeros

        fire_load(sa, wbase)
        wait_load(sa, wbase)
        fire_gather(sa)
        fire_load(sb, wbase + 1)

        def half(k, s, o, t_cur, t_nxt):
            wait_load(o, t_nxt)
            fire_gather(o)
            wait_gather(s)

            @pl.when(k > 0)
            def _():
                wait_store(s, t_cur - 2)

            fire_flags(s, t_cur)
            fire_load(s, t_nxt + 1)
            transpose_emb(s)
            wait_flags(s, t_cur)
            fire_store(s, t_cur)

        def step(k, carry):
            t = wbase + 2 * k
            half(k, sa, sb, t, t + 1)
            half(k, sb, sa, t + 1, t + 2)
            return carry

        lax.fori_loop(0, tiles_per_w // 2 - 1, step, 0)

        t_last = wbase + tiles_per_w - 1
        wait_gather(sa)
        wait_store(sa, t_last - 3)
        fire_flags(sa, t_last - 1)
        transpose_emb(sa)
        wait_flags(sa, t_last - 1)
        fire_store(sa, t_last - 1)
        wait_load(sb, t_last)
        fire_gather(sb)
        wait_gather(sb)
        wait_store(sb, t_last - 2)
        fire_flags(sb, t_last)
        transpose_emb(sb)
        wait_flags(sb, t_last)
        fire_store(sb, t_last)
        wait_store(sa, t_last - 1)
        wait_store(sb, t_last)

    scratch = (
        [pltpu.VMEM((8, 128), jnp.int32)]
        + [pltpu.VMEM((128, 8), jnp.float32) for _ in range(8)]
        + [pltpu.VMEM((_TR, 128), jnp.float32)]
        + [pltpu.VMEM((8, 128), jnp.int32)]
        + [pltpu.VMEM((128, 8), jnp.float32) for _ in range(8)]
        + [pltpu.VMEM((_TR, 128), jnp.float32)]
        + [pltpu.SemaphoreType.DMA] * 8
    )
    return pl.kernel(
        body,
        out_type=jax.ShapeDtypeStruct((nt * _TR, 128), jnp.float32),
        mesh=mesh,
        scratch_types=scratch,
        compiler_params=pltpu.CompilerParams(
            needs_layout_passes=False, use_tc_tiling_on_sc=False),
        name="sc_embed_gather",
    )


@functools.lru_cache(maxsize=2)
def _tc_dense(bsz, seq):
    grp = 32
    bblk = bsz // (grp * 128)

    def body(ct_ref, wa_ref, v8_ref, wf8_ref, bf8_ref, bcat_ref,
             gam_ref, bet_ref, out_ref):
        wf = wf8_ref[...]
        scale = jnp.sum(jnp.abs(wf)) / 25.0 + 1e-8
        wq = jnp.clip(jnp.round(wf / scale), -1.0, 1.0) * scale
        v8 = v8_ref[...]
        dn = (((0,), (0,)), ((), ()))
        mq = lax.dot_general(wq, v8, dn, preferred_element_type=jnp.float32)
        bfv = lax.dot_general(v8, bf8_ref[...], dn,
                              preferred_element_type=jnp.float32)
        bias = bcat_ref[...] + bfv
        gam = gam_ref[...]
        bet = bet_ref[...]
        wa_eff = jnp.concatenate([wa_ref[0:_FLAG_ROW], mq], axis=0)
        for g in range(grp):
            tile = ct_ref[g * _TR:(g + 1) * _TR, :]
            big = lax.dot_general(wa_eff, tile, dn,
                                  preferred_element_type=jnp.float32)
            big = big + bias
            gate = jax.nn.sigmoid(big[:64, :])
            z = gate * big[64:, :]
            mu = jnp.mean(z, axis=0, keepdims=True)
            d = z - mu
            var = jnp.mean(d * d, axis=0, keepdims=True)
            o = d * lax.rsqrt(var + 1e-5) * gam + bet
            out_ref[0, :, pl.ds(g * 128, 128)] = o

    return pl.pallas_call(
        body,
        grid=(seq, bblk),
        in_specs=[
            pl.BlockSpec((grp * _TR, 128), lambda i, j: (i * bblk + j, 0)),
            pl.BlockSpec((_TR, 128), lambda i, j: (0, 0)),
            pl.BlockSpec((8, 128), lambda i, j: (0, 0)),
            pl.BlockSpec((8, 8), lambda i, j: (0, 0)),
            pl.BlockSpec((8, 1), lambda i, j: (0, 0)),
            pl.BlockSpec((128, 1), lambda i, j: (0, 0)),
            pl.BlockSpec((64, 1), lambda i, j: (0, 0)),
            pl.BlockSpec((64, 1), lambda i, j: (0, 0)),
        ],
        out_specs=pl.BlockSpec((1, 64, grp * 128), lambda i, j: (i, 0, j)),
        out_shape=jax.ShapeDtypeStruct((seq, 64, bsz), jnp.float32),
        compiler_params=pltpu.CompilerParams(
            dimension_semantics=("arbitrary", "arbitrary"),
            fuse_transposed_lhs_in_matmul=True),
        name="tc_fused_dense",
    )


def kernel(page_hash, offset, cache_line, addr_alignment, stride, reuse_dist,
           locality_cluster, entropy, address_flags, tbl_ph, tbl_off, tbl_cl,
           tbl_aa, tbl_str, tbl_rd, tbl_lc, tbl_ent, W_flags, b_flags,
           W_gate, b_gate, W_tr, b_tr, gamma, beta):
    bsz, seq = page_hash.shape
    n = bsz * seq
    nt = n // 128
    idx_t = [jnp.transpose(a).reshape(n).astype(jnp.int32)
             for a in (page_hash, offset, cache_line, addr_alignment, stride,
                       reuse_dist, locality_cluster, entropy)]
    tbls = [tbl_ph, tbl_off, tbl_cl, tbl_aa, tbl_str, tbl_rd, tbl_lc, tbl_ent]
    tbls = [t if t.shape[1] == 8 else jnp.pad(t, ((0, 0), (0, 8 - t.shape[1])))
            for t in tbls]
    flags_t = jnp.transpose(address_flags, (1, 2, 0))
    comb2 = _sc_gather(n, bsz)(*idx_t, flags_t, *tbls)

    wcat_t = jnp.concatenate([W_gate, W_tr], axis=0).T
    wa = jnp.zeros((_TR, 128), jnp.float32).at[:_NFEAT].set(wcat_t[:_NFEAT])
    v8 = jnp.zeros((8, 128), jnp.float32).at[:5].set(wcat_t[_NFEAT:44])
    wf8 = jnp.zeros((8, 8), jnp.float32).at[:5, :5].set(W_flags)
    bf8 = jnp.zeros((8, 1), jnp.float32).at[:5, 0].set(b_flags)
    bcat = jnp.concatenate([b_gate, b_tr]).reshape(128, 1)
    outp = _tc_dense(bsz, seq)(comb2, wa, v8, wf8, bf8, bcat,
                               gamma.reshape(64, 1), beta.reshape(64, 1))
    return jnp.transpose(outp, (2, 0, 1))


# speedup vs baseline: 29.0592x; 1.0821x over previous
import functools

import jax
import jax.numpy as jnp
from jax import lax
from jax.experimental import pallas as pl
from jax.experimental.pallas import tpu as pltpu
from jax.experimental.pallas import tpu_sc as plsc

_D = (8, 4, 4, 3, 6, 5, 6, 3)
_OFF = (0, 8, 12, 16, 19, 25, 30, 36)
_NFEAT = 39
_FLAG_ROW = 40
_TR = 48
_NC, _NS = 2, 16
_NW = _NC * _NS


class _Slot:
    def __init__(self, ib, gb, tile, sem_l, sem_g, sem_s, sem_f):
        self.ib, self.gb, self.tile = ib, gb, tile
        self.sem_l, self.sem_g, self.sem_s, self.sem_f = (
            sem_l, sem_g, sem_s, sem_f)


@functools.lru_cache(maxsize=2)
def _sc_gather(n_tokens, bsz):
    nt = n_tokens // 128
    tiles_per_w = nt // _NW
    tpl = bsz // 128
    assert tiles_per_w % 2 == 0 and tiles_per_w >= 4
    mesh = plsc.VectorSubcoreMesh(
        core_axis_name="c", subcore_axis_name="s",
        num_cores=_NC, num_subcores=_NS)

    def body(i0, i1, i2, i3, i4, i5, i6, i7, flags_ref,
             t0, t1, t2, t3, t4, t5, t6, t7, out_ref,
             ib_a, ga0, ga1, ga2, ga3, ga4, ga5, ga6, ga7, tile_a,
             ib_b, gb0, gb1, gb2, gb3, gb4, gb5, gb6, gb7, tile_b,
             sem_la, sem_ga, sem_sa, sem_fa, sem_lb, sem_gb, sem_sb, sem_fb):
        idx_refs = (i0, i1, i2, i3, i4, i5, i6, i7)
        tbl_refs = (t0, t1, t2, t3, t4, t5, t6, t7)
        sa = _Slot(ib_a, (ga0, ga1, ga2, ga3, ga4, ga5, ga6, ga7),
                   tile_a, sem_la, sem_ga, sem_sa, sem_fa)
        sb = _Slot(ib_b, (gb0, gb1, gb2, gb3, gb4, gb5, gb6, gb7),
                   tile_b, sem_lb, sem_gb, sem_sb, sem_fb)

        wid = lax.axis_index("s") * _NC + lax.axis_index("c")
        wbase = wid * tiles_per_w
        iota = lax.iota(jnp.int32, 16)
        zeros = jnp.zeros((16,), jnp.float32)
        cols = [jnp.full((16,), k, jnp.int32) for k in range(8)]

        def _flags_src(gt):
            l = gt // tpl
            b0 = (gt % tpl) * 128
            return flags_ref.at[l, :, pl.ds(b0, 128)]

        def fire_load(s, gt):
            for i in range(8):
                pltpu.async_copy(
                    idx_refs[i].at[pl.ds(gt * 128, 128)], s.ib.at[i], s.sem_l)

        def wait_load(s, gt):
            for i in range(8):
                pltpu.make_async_copy(
                    idx_refs[i].at[pl.ds(gt * 128, 128)], s.ib.at[i],
                    s.sem_l).wait()

        def fire_flags(s, gt):
            pltpu.async_copy(
                _flags_src(gt), s.tile.at[pl.ds(_FLAG_ROW, 5), :], s.sem_f)

        def wait_flags(s, gt):
            pltpu.make_async_copy(
                _flags_src(gt), s.tile.at[pl.ds(_FLAG_ROW, 5), :],
                s.sem_f).wait()

        def fire_gather(s):
            for i in range(8):
                pltpu.async_copy(tbl_refs[i].at[s.ib.at[i]], s.gb[i], s.sem_g)

        def wait_gather(s):
            for i in range(8):
                pltpu.make_async_copy(
                    tbl_refs[i].at[s.ib.at[i]], s.gb[i], s.sem_g).wait()

        def transpose_emb(s):
            for lg in range(8):
                rows = iota + lg * 16
                for i in range(8):
                    for k in range(_D[i]):
                        v = plsc.load_gather(s.gb[i], [rows, cols[k]])
                        s.tile[_OFF[i] + k, pl.ds(lg * 16, 16)] = v

        def fire_store(s, gt):
            pltpu.async_copy(s.tile, out_ref.at[pl.ds(gt * _TR, _TR), :],
                             s.sem_s)

        def wait_store(s, gt_old):
            pltpu.make_async_copy(
                s.tile, out_ref.at[pl.ds(gt_old * _TR, _TR), :], s.sem_s
            ).wait()

        for s in (sa, sb):
            for r in (39, 45, 46, 47):
                for lg in range(8):
                    s.tile[r, pl.ds(lg * 16, 16)] = zeros

        fire_load(sa, wbase)
        wait_load(sa, wbase)
        fire_gather(sa)
        fire_load(sb, wbase + 1)

        def half(k, s, o, t_cur, t_nxt):
            wait_load(o, t_nxt)
            fire_gather(o)
            wait_gather(s)

            @pl.when(k > 0)
            def _():
                wait_store(s, t_cur - 2)

            fire_flags(s, t_cur)
            fire_load(s, t_nxt + 1)
            transpose_emb(s)
            wait_flags(s, t_cur)
            fire_store(s, t_cur)

        def step(k, carry):
            t = wbase + 2 * k
            half(k, sa, sb, t, t + 1)
            half(k, sb, sa, t + 1, t + 2)
            return carry

        lax.fori_loop(0, tiles_per_w // 2 - 1, step, 0)

        t_last = wbase + tiles_per_w - 1
        wait_gather(sa)
        wait_store(sa, t_last - 3)
        fire_flags(sa, t_last - 1)
        transpose_emb(sa)
        wait_flags(sa, t_last - 1)
        fire_store(sa, t_last - 1)
        wait_load(sb, t_last)
        fire_gather(sb)
        wait_gather(sb)
        wait_store(sb, t_last - 2)
        fire_flags(sb, t_last)
        transpose_emb(sb)
        wait_flags(sb, t_last)
        fire_store(sb, t_last)
        wait_store(sa, t_last - 1)
        wait_store(sb, t_last)

    scratch = (
        [pltpu.VMEM((8, 128), jnp.int32)]
        + [pltpu.VMEM((128, 8), jnp.float32) for _ in range(8)]
        + [pltpu.VMEM((_TR, 128), jnp.float32)]
        + [pltpu.VMEM((8, 128), jnp.int32)]
        + [pltpu.VMEM((128, 8), jnp.float32) for _ in range(8)]
        + [pltpu.VMEM((_TR, 128), jnp.float32)]
        + [pltpu.SemaphoreType.DMA] * 8
    )
    return pl.kernel(
        body,
        out_type=jax.ShapeDtypeStruct((nt * _TR, 128), jnp.float32),
        mesh=mesh,
        scratch_types=scratch,
        compiler_params=pltpu.CompilerParams(
            needs_layout_passes=False, use_tc_tiling_on_sc=False),
        name="sc_embed_gather",
    )


@functools.lru_cache(maxsize=2)
def _tc_dense(bsz, seq):
    grp = 64
    bblk = bsz // (grp * 128)

    def body(ct_ref, wa_ref, v8_ref, wf8_ref, bf8_ref, bcat_ref,
             gam_ref, bet_ref, out_ref):
        wf = wf8_ref[...]
        scale = jnp.sum(jnp.abs(wf)) / 25.0 + 1e-8
        wq = jnp.clip(jnp.round(wf / scale), -1.0, 1.0) * scale
        v8 = v8_ref[...]
        dn = (((0,), (0,)), ((), ()))
        mq = lax.dot_general(wq, v8, dn, preferred_element_type=jnp.float32)
        bfv = lax.dot_general(v8, bf8_ref[...], dn,
                              preferred_element_type=jnp.float32)
        bias = bcat_ref[...] + bfv
        gam = gam_ref[...]
        bet = bet_ref[...]
        wa_eff = jnp.concatenate([wa_ref[0:_FLAG_ROW], mq], axis=0)
        for g in range(grp):
            tile = ct_ref[g * _TR:(g + 1) * _TR, :]
            big = lax.dot_general(wa_eff, tile, dn,
                                  preferred_element_type=jnp.float32)
            big = big + bias
            gate = jax.nn.sigmoid(big[:64, :])
            z = gate * big[64:, :]
            mu = jnp.mean(z, axis=0, keepdims=True)
            d = z - mu
            var = jnp.mean(d * d, axis=0, keepdims=True)
            o = d * lax.rsqrt(var + 1e-5) * gam + bet
            out_ref[0, :, pl.ds(g * 128, 128)] = o

    return pl.pallas_call(
        body,
        grid=(seq, bblk),
        in_specs=[
            pl.BlockSpec((grp * _TR, 128), lambda i, j: (i * bblk + j, 0)),
            pl.BlockSpec((_TR, 128), lambda i, j: (0, 0)),
            pl.BlockSpec((8, 128), lambda i, j: (0, 0)),
            pl.BlockSpec((8, 8), lambda i, j: (0, 0)),
            pl.BlockSpec((8, 1), lambda i, j: (0, 0)),
            pl.BlockSpec((128, 1), lambda i, j: (0, 0)),
            pl.BlockSpec((64, 1), lambda i, j: (0, 0)),
            pl.BlockSpec((64, 1), lambda i, j: (0, 0)),
        ],
        out_specs=pl.BlockSpec((1, 64, grp * 128), lambda i, j: (i, 0, j)),
        out_shape=jax.ShapeDtypeStruct((seq, 64, bsz), jnp.float32),
        compiler_params=pltpu.CompilerParams(
            dimension_semantics=("arbitrary", "arbitrary"),
            fuse_transposed_lhs_in_matmul=True),
        name="tc_fused_dense",
    )


def kernel(page_hash, offset, cache_line, addr_alignment, stride, reuse_dist,
           locality_cluster, entropy, address_flags, tbl_ph, tbl_off, tbl_cl,
           tbl_aa, tbl_str, tbl_rd, tbl_lc, tbl_ent, W_flags, b_flags,
           W_gate, b_gate, W_tr, b_tr, gamma, beta):
    bsz, seq = page_hash.shape
    n = bsz * seq
    nt = n // 128
    idx_t = [jnp.transpose(a).reshape(n).astype(jnp.int32)
             for a in (page_hash, offset, cache_line, addr_alignment, stride,
                       reuse_dist, locality_cluster, entropy)]
    tbls = [tbl_ph, tbl_off, tbl_cl, tbl_aa, tbl_str, tbl_rd, tbl_lc, tbl_ent]
    tbls = [t if t.shape[1] == 8 else jnp.pad(t, ((0, 0), (0, 8 - t.shape[1])))
            for t in tbls]
    flags_t = jnp.transpose(address_flags, (1, 2, 0))
    comb2 = _sc_gather(n, bsz)(*idx_t, flags_t, *tbls)

    wcat_t = jnp.concatenate([W_gate, W_tr], axis=0).T
    wa = jnp.zeros((_TR, 128), jnp.float32).at[:_NFEAT].set(wcat_t[:_NFEAT])
    v8 = jnp.zeros((8, 128), jnp.float32).at[:5].set(wcat_t[_NFEAT:44])
    wf8 = jnp.zeros((8, 8), jnp.float32).at[:5, :5].set(W_flags)
    bf8 = jnp.zeros((8, 1), jnp.float32).at[:5, 0].set(b_flags)
    bcat = jnp.concatenate([b_gate, b_tr]).reshape(128, 1)
    outp = _tc_dense(bsz, seq)(comb2, wa, v8, wf8, bf8, bcat,
                               gamma.reshape(64, 1), beta.reshape(64, 1))
    return jnp.transpose(outp, (2, 0, 1))
